# baseline (speedup 1.0000x reference)
import jax
import jax.numpy as jnp
from jax.experimental import pallas as pl

N = 100000


def _sigmoid_body(x_ref, o_ref):
    o_ref[...] = jax.nn.sigmoid(x_ref[...])


def _graph_conv(x, src, dst, ew, W_rel, b_rel, W_root):
    msg = x[src] * ew[:, None]
    agg = jax.ops.segment_sum(msg, dst, num_segments=N)
    return agg @ W_rel.T + b_rel + x @ W_root.T


def kernel(x, edge_index, edge_weights, Wr0, br0, Wt0, Wr1, br1, Wt1, Wr2, br2, Wt2, Wr3, br3, Wt3, Wr4, br4, Wt4):
    src = edge_index[0]
    dst = edge_index[1]
    params = [(Wr0, br0, Wt0), (Wr1, br1, Wt1), (Wr2, br2, Wt2),
              (Wr3, br3, Wt3), (Wr4, br4, Wt4)]
    h = x
    for i, (Wr, br, Wt) in enumerate(params):
        h = _graph_conv(h, src, dst, edge_weights, Wr, br, Wt)
        if i < len(params) - 1:
            h = jax.nn.relu(h)
    z = h.reshape(800, 125)
    out = pl.pallas_call(
        _sigmoid_body,
        out_shape=jax.ShapeDtypeStruct((800, 125), jnp.float32),
    )(z)
    return out.reshape(N, 1)


# speedup vs baseline: 6.4410x; 6.4410x over previous
import functools

import jax
import jax.numpy as jnp
from jax import lax
from jax.experimental import pallas as pl
from jax.experimental.pallas import tpu as pltpu
from jax.experimental.pallas import tpu_sc as plsc

N = 100000
NP = 100096
E = 3200000
NSUB = 16
B = 1000
RPT = NP // NSUB
RB = 2000
GRID = N // RB

_MESH = plsc.VectorSubcoreMesh(core_axis_name="c", subcore_axis_name="s")

_SCRATCH = [
    pltpu.VMEM((B,), jnp.int32),
    pltpu.VMEM((B,), jnp.int32),
    pltpu.VMEM((B,), jnp.float32),
    pltpu.VMEM((B, 16), jnp.float32),
    pltpu.VMEM_SHARED((NP, 16), jnp.float32),
    pltpu.SemaphoreType.DMA,
]


def _zero_fill(rows_v):
    def zi(i, c):
        rows_v[i] = jnp.zeros((16,), jnp.float32)
        return c
    lax.fori_loop(0, B, zi, 0)


def _zero_agg(rows_v, agg_sh, sid):
    for k in range(RPT // B):
        off = pl.multiple_of(sid * RPT + k * B, 8)
        pltpu.sync_copy(rows_v, agg_sh.at[pl.ds(off, B)])
    off = pl.multiple_of(sid * RPT + (RPT // B) * B, 8)
    pltpu.sync_copy(rows_v.at[pl.ds(0, RPT % B)], agg_sh.at[pl.ds(off, RPT % B)])


def _edge_pass(table, src_hbm, dst_hbm, ew_hbm, agg_sh,
               src_v, dst_v, ew_v, rows_v, sem, base, nblk, row_off):
    def blk(b, carry):
        off = base + b * B
        pltpu.sync_copy(src_hbm.at[pl.ds(off, B)], src_v)
        pltpu.sync_copy(dst_hbm.at[pl.ds(off, B)], dst_v)
        pltpu.sync_copy(ew_hbm.at[pl.ds(off, B)], ew_v)
        if row_off is not None:
            off16 = jnp.broadcast_to(row_off, (16,)).astype(jnp.int32)

            def adj(i, c2):
                sl = pl.ds(i * 16, 16)
                src_v[sl] = src_v[sl] + off16
                return c2
            lax.fori_loop(0, B // 16, adj, 0, unroll=8)
        pltpu.async_copy(table.at[src_v], rows_v, sem).wait()

        def srow(i, c2):
            ewi = plsc.load_gather(ew_v, [jnp.broadcast_to(i, (16,)).astype(jnp.int32)])
            rows_v[i] = rows_v[i] * ewi
            return c2
        lax.fori_loop(0, B, srow, 0, unroll=8)
        pltpu.sync_copy(rows_v, agg_sh.at[dst_v], add=True)
        return carry
    lax.fori_loop(0, nblk, blk, 0)


_SC_PARAMS = pltpu.CompilerParams(needs_layout_passes=False,
                                  use_tc_tiling_on_sc=False)


@functools.partial(
    pl.kernel, mesh=_MESH,
    out_type=jax.ShapeDtypeStruct((2 * NP, 16), jnp.float32),
    scratch_types=_SCRATCH, compiler_params=_SC_PARAMS)
def _agg_split(table, src, dst, ew, out,
               src_v, dst_v, ew_v, rows_v, agg_sh, sem):
    cid = lax.axis_index("c")
    sid = lax.axis_index("s")
    _zero_fill(rows_v)
    _zero_agg(rows_v, agg_sh, sid)
    plsc.subcore_barrier()
    wid = cid * NSUB + sid
    _edge_pass(table, src, dst, ew, agg_sh, src_v, dst_v, ew_v, rows_v, sem,
               base=wid * (E // 32), nblk=(E // 32) // B, row_off=None)
    plsc.subcore_barrier()
    soff = pl.multiple_of(sid * RPT, 8)
    doff = pl.multiple_of(cid * NP + sid * RPT, 8)
    pltpu.sync_copy(agg_sh.at[pl.ds(soff, RPT)], out.at[pl.ds(doff, RPT)])


@functools.partial(
    pl.kernel, mesh=_MESH,
    out_type=jax.ShapeDtypeStruct((4 * NP, 16), jnp.float32),
    scratch_types=_SCRATCH, compiler_params=_SC_PARAMS)
def _agg_chunk(table, src, dst, ew, out,
               src_v, dst_v, ew_v, rows_v, agg_sh, sem):
    cid = lax.axis_index("c")
    sid = lax.axis_index("s")
    for rep in range(2):
        chunk = 2 * cid + rep
        _zero_fill(rows_v)
        _zero_agg(rows_v, agg_sh, sid)
        plsc.subcore_barrier()
        _edge_pass(table, src, dst, ew, agg_sh, src_v, dst_v, ew_v, rows_v,
                   sem, base=sid * (E // 16), nblk=(E // 16) // B,
                   row_off=chunk * NP)
        plsc.subcore_barrier()
        soff = pl.multiple_of(sid * RPT, 8)
        doff = pl.multiple_of(chunk * NP + sid * RPT, 8)
        pltpu.sync_copy(agg_sh.at[pl.ds(soff, RPT)], out.at[pl.ds(doff, RPT)])


def _l0_body(agg_r, xp_r, wr_r, wt_r, b_r, out_r):
    agg = agg_r[0] + agg_r[1]
    acc = jnp.dot(agg, wr_r[...], preferred_element_type=jnp.float32)
    acc = acc + jnp.dot(xp_r[...], wt_r[...], preferred_element_type=jnp.float32)
    acc = acc + b_r[...]
    h = jnp.maximum(acc, 0.0)
    for c in range(4):
        out_r[c] = h[:, 16 * c:16 * (c + 1)]


def _mid_body(agg_r, h_r, wr_r, wt_r, b_r, out_r):
    a64 = jnp.concatenate([agg_r[c] for c in range(4)], axis=1)
    h64 = jnp.concatenate([h_r[c] for c in range(4)], axis=1)
    acc = jnp.dot(a64, wr_r[...], preferred_element_type=jnp.float32)
    acc = acc + jnp.dot(h64, wt_r[...], preferred_element_type=jnp.float32)
    acc = acc + b_r[...]
    h = jnp.maximum(acc, 0.0)
    for c in range(4):
        out_r[c] = h[:, 16 * c:16 * (c + 1)]


def _l3_body(agg_r, h_r, wr_r, wt_r, b_r, wr4_r, wt4_r, yp_r, z_r):
    a64 = jnp.concatenate([agg_r[c] for c in range(4)], axis=1)
    h64 = jnp.concatenate([h_r[c] for c in range(4)], axis=1)
    acc = jnp.dot(a64, wr_r[...], preferred_element_type=jnp.float32)
    acc = acc + jnp.dot(h64, wt_r[...], preferred_element_type=jnp.float32)
    acc = acc + b_r[...]
    h4 = jnp.maximum(acc, 0.0)
    y = jnp.dot(h4, wr4_r[...], preferred_element_type=jnp.float32)
    z = jnp.dot(h4, wt4_r[...], preferred_element_type=jnp.float32)
    yp_r[...] = jnp.concatenate([y, jnp.zeros((RB, 15), jnp.float32)], axis=1)
    z_r[...] = z


def _fin_body(aggy_r, z_r, b_r, out_r):
    s = aggy_r[0][:, 0:1] + aggy_r[1][:, 0:1] + z_r[...] + b_r[...]
    out_r[...] = jax.nn.sigmoid(s)


def _chunk_spec():
    return pl.BlockSpec((4, RB, 16), lambda i: (0, i, 0))


def _pair_spec():
    return pl.BlockSpec((2, RB, 16), lambda i: (0, i, 0))


def _full(shape):
    return pl.BlockSpec(shape, lambda i: tuple(0 for _ in shape))


def _l0_call(agg2, xp, wr, wt, b):
    return pl.pallas_call(
        _l0_body,
        grid=(GRID,),
        in_specs=[_pair_spec(),
                  pl.BlockSpec((RB, 16), lambda i: (i, 0)),
                  _full((16, 64)), _full((16, 64)), _full((1, 64))],
        out_specs=_chunk_spec(),
        out_shape=jax.ShapeDtypeStruct((4, NP, 16), jnp.float32),
    )(agg2, xp, wr, wt, b)


def _mid_call(agg4, h4, wr, wt, b):
    return pl.pallas_call(
        _mid_body,
        grid=(GRID,),
        in_specs=[_chunk_spec(), _chunk_spec(),
                  _full((64, 64)), _full((64, 64)), _full((1, 64))],
        out_specs=_chunk_spec(),
        out_shape=jax.ShapeDtypeStruct((4, NP, 16), jnp.float32),
    )(agg4, h4, wr, wt, b)


def _l3_call(agg4, h4, wr, wt, b, wr4, wt4):
    return pl.pallas_call(
        _l3_body,
        grid=(GRID,),
        in_specs=[_chunk_spec(), _chunk_spec(),
                  _full((64, 64)), _full((64, 64)), _full((1, 64)),
                  _full((64, 1)), _full((64, 1))],
        out_specs=[pl.BlockSpec((RB, 16), lambda i: (i, 0)),
                   pl.BlockSpec((RB, 1), lambda i: (i, 0))],
        out_shape=[jax.ShapeDtypeStruct((N, 16), jnp.float32),
                   jax.ShapeDtypeStruct((N, 1), jnp.float32)],
    )(agg4, h4, wr, wt, b, wr4, wt4)


def _fin_call(aggy2, z, b4):
    return pl.pallas_call(
        _fin_body,
        grid=(GRID,),
        in_specs=[_pair_spec(),
                  pl.BlockSpec((RB, 1), lambda i: (i, 0)),
                  _full((1, 1))],
        out_specs=pl.BlockSpec((RB, 1), lambda i: (i, 0)),
        out_shape=jax.ShapeDtypeStruct((N, 1), jnp.float32),
    )(aggy2, z, b4)


def kernel(x, edge_index, edge_weights, Wr0, br0, Wt0, Wr1, br1, Wt1,
           Wr2, br2, Wt2, Wr3, br3, Wt3, Wr4, br4, Wt4):
    src = edge_index[0]
    dst = edge_index[1]
    ew = edge_weights

    x_pad = jnp.pad(x, ((0, 0), (0, 3)))
    wr0p = jnp.pad(Wr0, ((0, 0), (0, 3))).T
    wt0p = jnp.pad(Wt0, ((0, 0), (0, 3))).T

    agg0 = _agg_split(x_pad, src, dst, ew)
    h1 = _l0_call(agg0.reshape(2, NP, 16), x_pad, wr0p, wt0p,
                  br0.reshape(1, 64))

    h = h1
    for Wr, br, Wt in ((Wr1, br1, Wt1), (Wr2, br2, Wt2)):
        agg = _agg_chunk(h.reshape(4 * NP, 16), src, dst, ew)
        h = _mid_call(agg.reshape(4, NP, 16), h, Wr.T, Wt.T,
                      br.reshape(1, 64))

    agg3 = _agg_chunk(h.reshape(4 * NP, 16), src, dst, ew)
    y_pad, z = _l3_call(agg3.reshape(4, NP, 16), h, Wr3.T, Wt3.T,
                        br3.reshape(1, 64), Wr4.T, Wt4.T)

    aggy = _agg_split(y_pad, src, dst, ew)
    out = _fin_call(aggy.reshape(2, NP, 16), z, br4.reshape(1, 1))
    return out


# speedup vs baseline: 8.2740x; 1.2846x over previous
import functools

import jax
import jax.numpy as jnp
from jax import lax
from jax.experimental import pallas as pl
from jax.experimental.pallas import tpu as pltpu
from jax.experimental.pallas import tpu_sc as plsc

N = 100000
NP = 100096
E = 3200000
NSUB = 16
B = 800
RPT = NP // NSUB
RB = 2000
GRID = N // RB

_MESH = plsc.VectorSubcoreMesh(core_axis_name="c", subcore_axis_name="s")

_SCRATCH = [
    pltpu.VMEM((B,), jnp.int32),
    pltpu.VMEM((B,), jnp.int32),
    pltpu.VMEM((B,), jnp.float32),
    pltpu.VMEM((B, 16), jnp.float32),
    pltpu.VMEM((B,), jnp.int32),
    pltpu.VMEM((B,), jnp.int32),
    pltpu.VMEM((B,), jnp.float32),
    pltpu.VMEM((B, 16), jnp.float32),
    pltpu.VMEM_SHARED((NP, 16), jnp.float32),
    pltpu.SemaphoreType.DMA,
    pltpu.SemaphoreType.DMA,
    pltpu.SemaphoreType.DMA,
    pltpu.SemaphoreType.DMA,
    pltpu.SemaphoreType.DMA,
    pltpu.SemaphoreType.DMA,
]


def _zero_fill(rows_v):
    def zi(i, c):
        rows_v[i] = jnp.zeros((16,), jnp.float32)
        return c
    lax.fori_loop(0, B, zi, 0)


def _zero_agg(rows_v, agg_sh, sid):
    for k in range(RPT // B):
        off = pl.multiple_of(sid * RPT + k * B, 8)
        pltpu.sync_copy(rows_v, agg_sh.at[pl.ds(off, B)])
    rem = RPT % B
    if rem:
        off = pl.multiple_of(sid * RPT + (RPT // B) * B, 8)
        pltpu.sync_copy(rows_v.at[pl.ds(0, rem)], agg_sh.at[pl.ds(off, rem)])


def _scale(rows_v, ew_v):
    def srow(i, c):
        ewi = plsc.load_gather(
            ew_v, [jnp.broadcast_to(i, (16,)).astype(jnp.int32)])
        rows_v[i] = rows_v[i] * ewi
        return c
    lax.fori_loop(0, B, srow, 0, unroll=16)


def _edge_pass(tbl, src_hbm, dst_hbm, ew_hbm, agg_sh, bufs, base, nblk):
    def body(g, carry):
        offs = (base + (2 * g) * B, base + (2 * g) * B + B)
        eds = []
        for (sv, dv, ev, rv, se, sg, ss), off in zip(bufs, offs):
            d_s = pltpu.async_copy(src_hbm.at[pl.ds(off, B)], sv, se)
            d_d = pltpu.async_copy(dst_hbm.at[pl.ds(off, B)], dv, se)
            d_e = pltpu.async_copy(ew_hbm.at[pl.ds(off, B)], ev, se)
            eds.append((d_s, d_d, d_e))
        gds = []
        for (sv, dv, ev, rv, se, sg, ss), (d_s, d_d, d_e) in zip(bufs, eds):
            d_s.wait()
            gds.append(pltpu.async_copy(tbl.at[sv], rv, sg))
        sds = []
        for (sv, dv, ev, rv, se, sg, ss), (d_s, d_d, d_e), gd in zip(
                bufs, eds, gds):
            d_e.wait()
            gd.wait()
            _scale(rv, ev)
            d_d.wait()
            sds.append(pltpu.async_copy(rv, agg_sh.at[dv], ss, add=True))
        for sd in sds:
            sd.wait()
        return carry
    lax.fori_loop(0, nblk // 2, body, 0)
    if nblk % 2:
        off = base + (nblk - 1) * B
        (sv, dv, ev, rv, se, sg, ss) = bufs[0]
        pltpu.sync_copy(src_hbm.at[pl.ds(off, B)], sv)
        pltpu.sync_copy(dst_hbm.at[pl.ds(off, B)], dv)
        pltpu.sync_copy(ew_hbm.at[pl.ds(off, B)], ev)
        pltpu.async_copy(tbl.at[sv], rv, sg).wait()
        _scale(rv, ev)
        pltpu.sync_copy(rv, agg_sh.at[dv], add=True)


_SC_PARAMS = pltpu.CompilerParams(needs_layout_passes=False,
                                  use_tc_tiling_on_sc=False)


@functools.partial(
    pl.kernel, mesh=_MESH,
    out_type=jax.ShapeDtypeStruct((2 * NP, 16), jnp.float32),
    scratch_types=_SCRATCH, compiler_params=_SC_PARAMS)
def _agg_split(table, src, dst, ew, out,
               sv0, dv0, ev0, rv0, sv1, dv1, ev1, rv1, agg_sh,
               se0, sg0, ss0, se1, sg1, ss1):
    cid = lax.axis_index("c")
    sid = lax.axis_index("s")
    bufs = ((sv0, dv0, ev0, rv0, se0, sg0, ss0),
            (sv1, dv1, ev1, rv1, se1, sg1, ss1))
    _zero_fill(rv0)
    _zero_agg(rv0, agg_sh, sid)
    plsc.subcore_barrier()
    wid = cid * NSUB + sid
    _edge_pass(table, src, dst, ew, agg_sh, bufs,
               base=wid * (E // 32), nblk=(E // 32) // B)
    plsc.subcore_barrier()
    soff = pl.multiple_of(sid * RPT, 8)
    doff = pl.multiple_of(cid * NP + sid * RPT, 8)
    pltpu.sync_copy(agg_sh.at[pl.ds(soff, RPT)], out.at[pl.ds(doff, RPT)])


@functools.partial(
    pl.kernel, mesh=_MESH,
    out_type=jax.ShapeDtypeStruct((4 * NP, 16), jnp.float32),
    scratch_types=_SCRATCH, compiler_params=_SC_PARAMS)
def _agg_chunk(table, src, dst, ew, out,
               sv0, dv0, ev0, rv0, sv1, dv1, ev1, rv1, agg_sh,
               se0, sg0, ss0, se1, sg1, ss1):
    cid = lax.axis_index("c")
    sid = lax.axis_index("s")
    bufs = ((sv0, dv0, ev0, rv0, se0, sg0, ss0),
            (sv1, dv1, ev1, rv1, se1, sg1, ss1))
    for rep in range(2):
        chunk = 2 * cid + rep
        toff = pl.multiple_of(chunk * NP, 8)
        tbl = table.at[pl.ds(toff, NP)]
        _zero_fill(rv0)
        _zero_agg(rv0, agg_sh, sid)
        plsc.subcore_barrier()
        _edge_pass(tbl, src, dst, ew, agg_sh, bufs,
                   base=sid * (E // 16), nblk=(E // 16) // B)
        plsc.subcore_barrier()
        soff = pl.multiple_of(sid * RPT, 8)
        doff = pl.multiple_of(chunk * NP + sid * RPT, 8)
        pltpu.sync_copy(agg_sh.at[pl.ds(soff, RPT)], out.at[pl.ds(doff, RPT)])


def _l0_body(agg_r, xp_r, wr_r, wt_r, b_r, out_r):
    agg = agg_r[0] + agg_r[1]
    acc = jnp.dot(agg, wr_r[...], preferred_element_type=jnp.float32)
    acc = acc + jnp.dot(xp_r[...], wt_r[...], preferred_element_type=jnp.float32)
    acc = acc + b_r[...]
    h = jnp.maximum(acc, 0.0)
    for c in range(4):
        out_r[c] = h[:, 16 * c:16 * (c + 1)]


def _mid_body(agg_r, h_r, wr_r, wt_r, b_r, out_r):
    a64 = jnp.concatenate([agg_r[c] for c in range(4)], axis=1)
    h64 = jnp.concatenate([h_r[c] for c in range(4)], axis=1)
    acc = jnp.dot(a64, wr_r[...], preferred_element_type=jnp.float32)
    acc = acc + jnp.dot(h64, wt_r[...], preferred_element_type=jnp.float32)
    acc = acc + b_r[...]
    h = jnp.maximum(acc, 0.0)
    for c in range(4):
        out_r[c] = h[:, 16 * c:16 * (c + 1)]


def _l3_body(agg_r, h_r, wr_r, wt_r, b_r, wr4_r, wt4_r, yp_r, z_r):
    a64 = jnp.concatenate([agg_r[c] for c in range(4)], axis=1)
    h64 = jnp.concatenate([h_r[c] for c in range(4)], axis=1)
    acc = jnp.dot(a64, wr_r[...], preferred_element_type=jnp.float32)
    acc = acc + jnp.dot(h64, wt_r[...], preferred_element_type=jnp.float32)
    acc = acc + b_r[...]
    h4 = jnp.maximum(acc, 0.0)
    y = jnp.dot(h4, wr4_r[...], preferred_element_type=jnp.float32)
    z = jnp.dot(h4, wt4_r[...], preferred_element_type=jnp.float32)
    yp_r[...] = jnp.concatenate([y, jnp.zeros((RB, 15), jnp.float32)], axis=1)
    z_r[...] = z


def _fin_body(aggy_r, z_r, b_r, out_r):
    s = aggy_r[0][:, 0:1] + aggy_r[1][:, 0:1] + z_r[...] + b_r[...]
    out_r[...] = jax.nn.sigmoid(s)


def _chunk_spec():
    return pl.BlockSpec((4, RB, 16), lambda i: (0, i, 0))


def _pair_spec():
    return pl.BlockSpec((2, RB, 16), lambda i: (0, i, 0))


def _full(shape):
    return pl.BlockSpec(shape, lambda i: tuple(0 for _ in shape))


def _l0_call(agg2, xp, wr, wt, b):
    return pl.pallas_call(
        _l0_body,
        grid=(GRID,),
        in_specs=[_pair_spec(),
                  pl.BlockSpec((RB, 16), lambda i: (i, 0)),
                  _full((16, 64)), _full((16, 64)), _full((1, 64))],
        out_specs=_chunk_spec(),
        out_shape=jax.ShapeDtypeStruct((4, NP, 16), jnp.float32),
    )(agg2, xp, wr, wt, b)


def _mid_call(agg4, h4, wr, wt, b):
    return pl.pallas_call(
        _mid_body,
        grid=(GRID,),
        in_specs=[_chunk_spec(), _chunk_spec(),
                  _full((64, 64)), _full((64, 64)), _full((1, 64))],
        out_specs=_chunk_spec(),
        out_shape=jax.ShapeDtypeStruct((4, NP, 16), jnp.float32),
    )(agg4, h4, wr, wt, b)


def _l3_call(agg4, h4, wr, wt, b, wr4, wt4):
    return pl.pallas_call(
        _l3_body,
        grid=(GRID,),
        in_specs=[_chunk_spec(), _chunk_spec(),
                  _full((64, 64)), _full((64, 64)), _full((1, 64)),
                  _full((64, 1)), _full((64, 1))],
        out_specs=[pl.BlockSpec((RB, 16), lambda i: (i, 0)),
                   pl.BlockSpec((RB, 1), lambda i: (i, 0))],
        out_shape=[jax.ShapeDtypeStruct((N, 16), jnp.float32),
                   jax.ShapeDtypeStruct((N, 1), jnp.float32)],
    )(agg4, h4, wr, wt, b, wr4, wt4)


def _fin_call(aggy2, z, b4):
    return pl.pallas_call(
        _fin_body,
        grid=(GRID,),
        in_specs=[_pair_spec(),
                  pl.BlockSpec((RB, 1), lambda i: (i, 0)),
                  _full((1, 1))],
        out_specs=pl.BlockSpec((RB, 1), lambda i: (i, 0)),
        out_shape=jax.ShapeDtypeStruct((N, 1), jnp.float32),
    )(aggy2, z, b4)


def kernel(x, edge_index, edge_weights, Wr0, br0, Wt0, Wr1, br1, Wt1,
           Wr2, br2, Wt2, Wr3, br3, Wt3, Wr4, br4, Wt4):
    src = edge_index[0]
    dst = edge_index[1]
    ew = edge_weights

    x_pad = jnp.pad(x, ((0, 0), (0, 3)))
    wr0p = jnp.pad(Wr0, ((0, 0), (0, 3))).T
    wt0p = jnp.pad(Wt0, ((0, 0), (0, 3))).T

    agg0 = _agg_split(x_pad, src, dst, ew)
    h1 = _l0_call(agg0.reshape(2, NP, 16), x_pad, wr0p, wt0p,
                  br0.reshape(1, 64))

    h = h1
    for Wr, br, Wt in ((Wr1, br1, Wt1), (Wr2, br2, Wt2)):
        agg = _agg_chunk(h.reshape(4 * NP, 16), src, dst, ew)
        h = _mid_call(agg.reshape(4, NP, 16), h, Wr.T, Wt.T,
                      br.reshape(1, 64))

    agg3 = _agg_chunk(h.reshape(4 * NP, 16), src, dst, ew)
    y_pad, z = _l3_call(agg3.reshape(4, NP, 16), h, Wr3.T, Wt3.T,
                        br3.reshape(1, 64), Wr4.T, Wt4.T)

    aggy = _agg_split(y_pad, src, dst, ew)
    out = _fin_call(aggy.reshape(2, NP, 16), z, br4.reshape(1, 1))
    return out


# speedup vs baseline: 15.8540x; 1.9161x over previous
import functools

import jax
import jax.numpy as jnp
from jax import lax
from jax.experimental import pallas as pl
from jax.experimental.pallas import tpu as pltpu
from jax.experimental.pallas import tpu_sc as plsc

N = 100000
NP = 100096
E = 3200000
NSUB = 16
B = 800
RPT = NP // NSUB
RB = 2000
GRID = N // RB

_MESH = plsc.VectorSubcoreMesh(core_axis_name="c", subcore_axis_name="s")

_SCRATCH = [
    pltpu.VMEM((B,), jnp.int32),
    pltpu.VMEM((B,), jnp.int32),
    pltpu.VMEM((B,), jnp.float32),
    pltpu.VMEM((B, 16), jnp.float32),
    pltpu.VMEM((B,), jnp.int32),
    pltpu.VMEM((B,), jnp.int32),
    pltpu.VMEM((B,), jnp.float32),
    pltpu.VMEM((B, 16), jnp.float32),
    pltpu.VMEM_SHARED((NP, 16), jnp.float32),
    pltpu.SemaphoreType.DMA,
    pltpu.SemaphoreType.DMA,
    pltpu.SemaphoreType.DMA,
    pltpu.SemaphoreType.DMA,
    pltpu.SemaphoreType.DMA,
    pltpu.SemaphoreType.DMA,
]


def _zero_fill(rows_v):
    def zi(i, c):
        rows_v[i] = jnp.zeros((16,), jnp.float32)
        return c
    lax.fori_loop(0, B, zi, 0)


def _zero_agg(rows_v, agg_sh, sid):
    for k in range(RPT // B):
        off = pl.multiple_of(sid * RPT + k * B, 8)
        pltpu.sync_copy(rows_v, agg_sh.at[pl.ds(off, B)])
    rem = RPT % B
    if rem:
        off = pl.multiple_of(sid * RPT + (RPT // B) * B, 8)
        pltpu.sync_copy(rows_v.at[pl.ds(0, rem)], agg_sh.at[pl.ds(off, rem)])


def _scale(rows_v, ew_v):
    @plsc.parallel_loop(0, B, unroll=8)
    def _srow(i):
        ewi = plsc.load_gather(
            ew_v, [jnp.broadcast_to(i, (16,)).astype(jnp.int32)])
        rows_v[i] = rows_v[i] * ewi


def _edge_pass(tbl, src_hbm, dst_hbm, ew_hbm, agg_sh, bufs, base, nblk):
    def body(g, carry):
        offs = (base + (2 * g) * B, base + (2 * g) * B + B)
        eds = []
        for (sv, dv, ev, rv, se, sg, ss), off in zip(bufs, offs):
            d_s = pltpu.async_copy(src_hbm.at[pl.ds(off, B)], sv, se)
            d_d = pltpu.async_copy(dst_hbm.at[pl.ds(off, B)], dv, se)
            d_e = pltpu.async_copy(ew_hbm.at[pl.ds(off, B)], ev, se)
            eds.append((d_s, d_d, d_e))
        gds = []
        for (sv, dv, ev, rv, se, sg, ss), (d_s, d_d, d_e) in zip(bufs, eds):
            d_s.wait()
            gds.append(pltpu.async_copy(tbl.at[sv], rv, sg))
        sds = []
        for (sv, dv, ev, rv, se, sg, ss), (d_s, d_d, d_e), gd in zip(
                bufs, eds, gds):
            d_e.wait()
            gd.wait()
            _scale(rv, ev)
            d_d.wait()
            sds.append(pltpu.async_copy(rv, agg_sh.at[dv], ss, add=True))
        for sd in sds:
            sd.wait()
        return carry
    lax.fori_loop(0, nblk // 2, body, 0)
    if nblk % 2:
        off = base + (nblk - 1) * B
        (sv, dv, ev, rv, se, sg, ss) = bufs[0]
        pltpu.sync_copy(src_hbm.at[pl.ds(off, B)], sv)
        pltpu.sync_copy(dst_hbm.at[pl.ds(off, B)], dv)
        pltpu.sync_copy(ew_hbm.at[pl.ds(off, B)], ev)
        pltpu.async_copy(tbl.at[sv], rv, sg).wait()
        _scale(rv, ev)
        pltpu.sync_copy(rv, agg_sh.at[dv], add=True)


_SC_PARAMS = pltpu.CompilerParams(needs_layout_passes=False,
                                  use_tc_tiling_on_sc=False)


@functools.partial(
    pl.kernel, mesh=_MESH,
    out_type=jax.ShapeDtypeStruct((2 * NP, 16), jnp.float32),
    scratch_types=_SCRATCH, compiler_params=_SC_PARAMS)
def _agg_split(table, src, dst, ew, out,
               sv0, dv0, ev0, rv0, sv1, dv1, ev1, rv1, agg_sh,
               se0, sg0, ss0, se1, sg1, ss1):
    cid = lax.axis_index("c")
    sid = lax.axis_index("s")
    bufs = ((sv0, dv0, ev0, rv0, se0, sg0, ss0),
            (sv1, dv1, ev1, rv1, se1, sg1, ss1))
    _zero_fill(rv0)
    _zero_agg(rv0, agg_sh, sid)
    plsc.subcore_barrier()
    wid = cid * NSUB + sid
    _edge_pass(table, src, dst, ew, agg_sh, bufs,
               base=wid * (E // 32), nblk=(E // 32) // B)
    plsc.subcore_barrier()
    soff = pl.multiple_of(sid * RPT, 8)
    doff = pl.multiple_of(cid * NP + sid * RPT, 8)
    pltpu.sync_copy(agg_sh.at[pl.ds(soff, RPT)], out.at[pl.ds(doff, RPT)])


@functools.partial(
    pl.kernel, mesh=_MESH,
    out_type=jax.ShapeDtypeStruct((4 * NP, 16), jnp.float32),
    scratch_types=_SCRATCH, compiler_params=_SC_PARAMS)
def _agg_chunk(table, src, dst, ew, out,
               sv0, dv0, ev0, rv0, sv1, dv1, ev1, rv1, agg_sh,
               se0, sg0, ss0, se1, sg1, ss1):
    cid = lax.axis_index("c")
    sid = lax.axis_index("s")
    bufs = ((sv0, dv0, ev0, rv0, se0, sg0, ss0),
            (sv1, dv1, ev1, rv1, se1, sg1, ss1))
    for rep in range(2):
        chunk = 2 * cid + rep
        toff = pl.multiple_of(chunk * NP, 8)
        tbl = table.at[pl.ds(toff, NP)]
        _zero_fill(rv0)
        _zero_agg(rv0, agg_sh, sid)
        plsc.subcore_barrier()
        _edge_pass(tbl, src, dst, ew, agg_sh, bufs,
                   base=sid * (E // 16), nblk=(E // 16) // B)
        plsc.subcore_barrier()
        soff = pl.multiple_of(sid * RPT, 8)
        doff = pl.multiple_of(chunk * NP + sid * RPT, 8)
        pltpu.sync_copy(agg_sh.at[pl.ds(soff, RPT)], out.at[pl.ds(doff, RPT)])


def _l0_body(agg_r, xp_r, wr_r, wt_r, b_r, out_r):
    agg = agg_r[0] + agg_r[1]
    acc = jnp.dot(agg, wr_r[...], preferred_element_type=jnp.float32)
    acc = acc + jnp.dot(xp_r[...], wt_r[...], preferred_element_type=jnp.float32)
    acc = acc + b_r[...]
    h = jnp.maximum(acc, 0.0)
    for c in range(4):
        out_r[c] = h[:, 16 * c:16 * (c + 1)]


def _mid_body(agg_r, h_r, wr_r, wt_r, b_r, out_r):
    a64 = jnp.concatenate([agg_r[c] for c in range(4)], axis=1)
    h64 = jnp.concatenate([h_r[c] for c in range(4)], axis=1)
    acc = jnp.dot(a64, wr_r[...], preferred_element_type=jnp.float32)
    acc = acc + jnp.dot(h64, wt_r[...], preferred_element_type=jnp.float32)
    acc = acc + b_r[...]
    h = jnp.maximum(acc, 0.0)
    for c in range(4):
        out_r[c] = h[:, 16 * c:16 * (c + 1)]


def _l3_body(agg_r, h_r, wr_r, wt_r, b_r, wr4_r, wt4_r, yp_r, z_r):
    a64 = jnp.concatenate([agg_r[c] for c in range(4)], axis=1)
    h64 = jnp.concatenate([h_r[c] for c in range(4)], axis=1)
    acc = jnp.dot(a64, wr_r[...], preferred_element_type=jnp.float32)
    acc = acc + jnp.dot(h64, wt_r[...], preferred_element_type=jnp.float32)
    acc = acc + b_r[...]
    h4 = jnp.maximum(acc, 0.0)
    y = jnp.dot(h4, wr4_r[...], preferred_element_type=jnp.float32)
    z = jnp.dot(h4, wt4_r[...], preferred_element_type=jnp.float32)
    yp_r[...] = jnp.concatenate([y, jnp.zeros((RB, 15), jnp.float32)], axis=1)
    z_r[...] = z


def _fin_body(aggy_r, z_r, b_r, out_r):
    s = aggy_r[0][:, 0:1] + aggy_r[1][:, 0:1] + z_r[...] + b_r[...]
    out_r[...] = jax.nn.sigmoid(s)


def _chunk_spec():
    return pl.BlockSpec((4, RB, 16), lambda i: (0, i, 0))


def _pair_spec():
    return pl.BlockSpec((2, RB, 16), lambda i: (0, i, 0))


def _full(shape):
    return pl.BlockSpec(shape, lambda i: tuple(0 for _ in shape))


def _l0_call(agg2, xp, wr, wt, b):
    return pl.pallas_call(
        _l0_body,
        grid=(GRID,),
        in_specs=[_pair_spec(),
                  pl.BlockSpec((RB, 16), lambda i: (i, 0)),
                  _full((16, 64)), _full((16, 64)), _full((1, 64))],
        out_specs=_chunk_spec(),
        out_shape=jax.ShapeDtypeStruct((4, NP, 16), jnp.float32),
    )(agg2, xp, wr, wt, b)


def _mid_call(agg4, h4, wr, wt, b):
    return pl.pallas_call(
        _mid_body,
        grid=(GRID,),
        in_specs=[_chunk_spec(), _chunk_spec(),
                  _full((64, 64)), _full((64, 64)), _full((1, 64))],
        out_specs=_chunk_spec(),
        out_shape=jax.ShapeDtypeStruct((4, NP, 16), jnp.float32),
    )(agg4, h4, wr, wt, b)


def _l3_call(agg4, h4, wr, wt, b, wr4, wt4):
    return pl.pallas_call(
        _l3_body,
        grid=(GRID,),
        in_specs=[_chunk_spec(), _chunk_spec(),
                  _full((64, 64)), _full((64, 64)), _full((1, 64)),
                  _full((64, 1)), _full((64, 1))],
        out_specs=[pl.BlockSpec((RB, 16), lambda i: (i, 0)),
                   pl.BlockSpec((RB, 1), lambda i: (i, 0))],
        out_shape=[jax.ShapeDtypeStruct((N, 16), jnp.float32),
                   jax.ShapeDtypeStruct((N, 1), jnp.float32)],
    )(agg4, h4, wr, wt, b, wr4, wt4)


def _fin_call(aggy2, z, b4):
    return pl.pallas_call(
        _fin_body,
        grid=(GRID,),
        in_specs=[_pair_spec(),
                  pl.BlockSpec((RB, 1), lambda i: (i, 0)),
                  _full((1, 1))],
        out_specs=pl.BlockSpec((RB, 1), lambda i: (i, 0)),
        out_shape=jax.ShapeDtypeStruct((N, 1), jnp.float32),
    )(aggy2, z, b4)


def kernel(x, edge_index, edge_weights, Wr0, br0, Wt0, Wr1, br1, Wt1,
           Wr2, br2, Wt2, Wr3, br3, Wt3, Wr4, br4, Wt4):
    src = edge_index[0]
    dst = edge_index[1]
    ew = edge_weights

    x_pad = jnp.pad(x, ((0, 0), (0, 3)))
    wr0p = jnp.pad(Wr0, ((0, 0), (0, 3))).T
    wt0p = jnp.pad(Wt0, ((0, 0), (0, 3))).T

    agg0 = _agg_split(x_pad, src, dst, ew)
    h1 = _l0_call(agg0.reshape(2, NP, 16), x_pad, wr0p, wt0p,
                  br0.reshape(1, 64))

    h = h1
    for Wr, br, Wt in ((Wr1, br1, Wt1), (Wr2, br2, Wt2)):
        agg = _agg_chunk(h.reshape(4 * NP, 16), src, dst, ew)
        h = _mid_call(agg.reshape(4, NP, 16), h, Wr.T, Wt.T,
                      br.reshape(1, 64))

    agg3 = _agg_chunk(h.reshape(4 * NP, 16), src, dst, ew)
    y_pad, z = _l3_call(agg3.reshape(4, NP, 16), h, Wr3.T, Wt3.T,
                        br3.reshape(1, 64), Wr4.T, Wt4.T)

    aggy = _agg_split(y_pad, src, dst, ew)
    out = _fin_call(aggy.reshape(2, NP, 16), z, br4.reshape(1, 1))
    return out


# speedup vs baseline: 15.9487x; 1.0060x over previous
import functools

import jax
import jax.numpy as jnp
from jax import lax
from jax.experimental import pallas as pl
from jax.experimental.pallas import tpu as pltpu
from jax.experimental.pallas import tpu_sc as plsc

N = 100000
NP = 100096
E = 3200000
NSUB = 16
B = 800
RPT = NP // NSUB
RB = 4000
GRID = N // RB

_MESH = plsc.VectorSubcoreMesh(core_axis_name="c", subcore_axis_name="s")

_SCRATCH = [
    pltpu.VMEM((B,), jnp.int32),
    pltpu.VMEM((B,), jnp.int32),
    pltpu.VMEM((B,), jnp.float32),
    pltpu.VMEM((B, 16), jnp.float32),
    pltpu.VMEM((B,), jnp.int32),
    pltpu.VMEM((B,), jnp.int32),
    pltpu.VMEM((B,), jnp.float32),
    pltpu.VMEM((B, 16), jnp.float32),
    pltpu.VMEM_SHARED((NP, 16), jnp.float32),
    pltpu.SemaphoreType.DMA,
    pltpu.SemaphoreType.DMA,
    pltpu.SemaphoreType.DMA,
    pltpu.SemaphoreType.DMA,
    pltpu.SemaphoreType.DMA,
    pltpu.SemaphoreType.DMA,
]


def _zero_fill(rows_v):
    def zi(i, c):
        rows_v[i] = jnp.zeros((16,), jnp.float32)
        return c
    lax.fori_loop(0, B, zi, 0)


def _zero_agg(rows_v, agg_sh, sid):
    for k in range(RPT // B):
        off = pl.multiple_of(sid * RPT + k * B, 8)
        pltpu.sync_copy(rows_v, agg_sh.at[pl.ds(off, B)])
    rem = RPT % B
    if rem:
        off = pl.multiple_of(sid * RPT + (RPT // B) * B, 8)
        pltpu.sync_copy(rows_v.at[pl.ds(0, rem)], agg_sh.at[pl.ds(off, rem)])


def _scale(rows_v, ew_v):
    @plsc.parallel_loop(0, B, unroll=8)
    def _srow(i):
        ewi = plsc.load_gather(
            ew_v, [jnp.broadcast_to(i, (16,)).astype(jnp.int32)])
        rows_v[i] = rows_v[i] * ewi


def _edge_pass(tbl, src_hbm, dst_hbm, ew_hbm, agg_sh, bufs, base, nblk):
    def body(g, carry):
        offs = (base + (2 * g) * B, base + (2 * g) * B + B)
        eds = []
        for (sv, dv, ev, rv, se, sg, ss), off in zip(bufs, offs):
            d_s = pltpu.async_copy(src_hbm.at[pl.ds(off, B)], sv, se)
            d_d = pltpu.async_copy(dst_hbm.at[pl.ds(off, B)], dv, se)
            d_e = pltpu.async_copy(ew_hbm.at[pl.ds(off, B)], ev, se)
            eds.append((d_s, d_d, d_e))
        gds = []
        for (sv, dv, ev, rv, se, sg, ss), (d_s, d_d, d_e) in zip(bufs, eds):
            d_s.wait()
            gds.append(pltpu.async_copy(tbl.at[sv], rv, sg))
        sds = []
        for (sv, dv, ev, rv, se, sg, ss), (d_s, d_d, d_e), gd in zip(
                bufs, eds, gds):
            d_e.wait()
            gd.wait()
            _scale(rv, ev)
            d_d.wait()
            sds.append(pltpu.async_copy(rv, agg_sh.at[dv], ss, add=True))
        for sd in sds:
            sd.wait()
        return carry
    lax.fori_loop(0, nblk // 2, body, 0)
    if nblk % 2:
        off = base + (nblk - 1) * B
        (sv, dv, ev, rv, se, sg, ss) = bufs[0]
        pltpu.sync_copy(src_hbm.at[pl.ds(off, B)], sv)
        pltpu.sync_copy(dst_hbm.at[pl.ds(off, B)], dv)
        pltpu.sync_copy(ew_hbm.at[pl.ds(off, B)], ev)
        pltpu.async_copy(tbl.at[sv], rv, sg).wait()
        _scale(rv, ev)
        pltpu.sync_copy(rv, agg_sh.at[dv], add=True)


_SC_PARAMS = pltpu.CompilerParams(needs_layout_passes=False,
                                  use_tc_tiling_on_sc=False)


@functools.partial(
    pl.kernel, mesh=_MESH,
    out_type=jax.ShapeDtypeStruct((2 * NP, 16), jnp.float32),
    scratch_types=_SCRATCH, compiler_params=_SC_PARAMS)
def _agg_split(table, src, dst, ew, out,
               sv0, dv0, ev0, rv0, sv1, dv1, ev1, rv1, agg_sh,
               se0, sg0, ss0, se1, sg1, ss1):
    cid = lax.axis_index("c")
    sid = lax.axis_index("s")
    bufs = ((sv0, dv0, ev0, rv0, se0, sg0, ss0),
            (sv1, dv1, ev1, rv1, se1, sg1, ss1))
    _zero_fill(rv0)
    _zero_agg(rv0, agg_sh, sid)
    plsc.subcore_barrier()
    wid = cid * NSUB + sid
    _edge_pass(table, src, dst, ew, agg_sh, bufs,
               base=wid * (E // 32), nblk=(E // 32) // B)
    plsc.subcore_barrier()
    soff = pl.multiple_of(sid * RPT, 8)
    doff = pl.multiple_of(cid * NP + sid * RPT, 8)
    pltpu.sync_copy(agg_sh.at[pl.ds(soff, RPT)], out.at[pl.ds(doff, RPT)])


@functools.partial(
    pl.kernel, mesh=_MESH,
    out_type=jax.ShapeDtypeStruct((4 * NP, 16), jnp.float32),
    scratch_types=_SCRATCH, compiler_params=_SC_PARAMS)
def _agg_chunk(table, src, dst, ew, out,
               sv0, dv0, ev0, rv0, sv1, dv1, ev1, rv1, agg_sh,
               se0, sg0, ss0, se1, sg1, ss1):
    cid = lax.axis_index("c")
    sid = lax.axis_index("s")
    bufs = ((sv0, dv0, ev0, rv0, se0, sg0, ss0),
            (sv1, dv1, ev1, rv1, se1, sg1, ss1))
    for rep in range(2):
        chunk = 2 * cid + rep
        toff = pl.multiple_of(chunk * NP, 8)
        tbl = table.at[pl.ds(toff, NP)]
        _zero_fill(rv0)
        _zero_agg(rv0, agg_sh, sid)
        plsc.subcore_barrier()
        _edge_pass(tbl, src, dst, ew, agg_sh, bufs,
                   base=sid * (E // 16), nblk=(E // 16) // B)
        plsc.subcore_barrier()
        soff = pl.multiple_of(sid * RPT, 8)
        doff = pl.multiple_of(chunk * NP + sid * RPT, 8)
        pltpu.sync_copy(agg_sh.at[pl.ds(soff, RPT)], out.at[pl.ds(doff, RPT)])


def _l0_body(agg_r, xp_r, wr_r, wt_r, b_r, out_r):
    agg = agg_r[0] + agg_r[1]
    acc = jnp.dot(agg, wr_r[...], preferred_element_type=jnp.float32)
    acc = acc + jnp.dot(xp_r[...], wt_r[...], preferred_element_type=jnp.float32)
    acc = acc + b_r[...]
    h = jnp.maximum(acc, 0.0)
    for c in range(4):
        out_r[c] = h[:, 16 * c:16 * (c + 1)]


def _mid_body(agg_r, h_r, wr_r, wt_r, b_r, out_r):
    a64 = jnp.concatenate([agg_r[c] for c in range(4)], axis=1)
    h64 = jnp.concatenate([h_r[c] for c in range(4)], axis=1)
    acc = jnp.dot(a64, wr_r[...], preferred_element_type=jnp.float32)
    acc = acc + jnp.dot(h64, wt_r[...], preferred_element_type=jnp.float32)
    acc = acc + b_r[...]
    h = jnp.maximum(acc, 0.0)
    for c in range(4):
        out_r[c] = h[:, 16 * c:16 * (c + 1)]


def _l3_body(agg_r, h_r, wr_r, wt_r, b_r, wr4_r, wt4_r, yp_r, z_r):
    a64 = jnp.concatenate([agg_r[c] for c in range(4)], axis=1)
    h64 = jnp.concatenate([h_r[c] for c in range(4)], axis=1)
    acc = jnp.dot(a64, wr_r[...], preferred_element_type=jnp.float32)
    acc = acc + jnp.dot(h64, wt_r[...], preferred_element_type=jnp.float32)
    acc = acc + b_r[...]
    h4 = jnp.maximum(acc, 0.0)
    y = jnp.dot(h4, wr4_r[...], preferred_element_type=jnp.float32)
    z = jnp.dot(h4, wt4_r[...], preferred_element_type=jnp.float32)
    yp_r[...] = jnp.concatenate([y, jnp.zeros((RB, 15), jnp.float32)], axis=1)
    z_r[...] = z


def _fin_body(aggy_r, z_r, b_r, out_r):
    s = aggy_r[0][:, 0:1] + aggy_r[1][:, 0:1] + z_r[...] + b_r[...]
    out_r[...] = jax.nn.sigmoid(s)


def _chunk_spec():
    return pl.BlockSpec((4, RB, 16), lambda i: (0, i, 0))


def _pair_spec():
    return pl.BlockSpec((2, RB, 16), lambda i: (0, i, 0))


def _full(shape):
    return pl.BlockSpec(shape, lambda i: tuple(0 for _ in shape))


def _l0_call(agg2, xp, wr, wt, b):
    return pl.pallas_call(
        _l0_body,
        grid=(GRID,),
        in_specs=[_pair_spec(),
                  pl.BlockSpec((RB, 16), lambda i: (i, 0)),
                  _full((16, 64)), _full((16, 64)), _full((1, 64))],
        out_specs=_chunk_spec(),
        out_shape=jax.ShapeDtypeStruct((4, NP, 16), jnp.float32),
    )(agg2, xp, wr, wt, b)


def _mid_call(agg4, h4, wr, wt, b):
    return pl.pallas_call(
        _mid_body,
        grid=(GRID,),
        in_specs=[_chunk_spec(), _chunk_spec(),
                  _full((64, 64)), _full((64, 64)), _full((1, 64))],
        out_specs=_chunk_spec(),
        out_shape=jax.ShapeDtypeStruct((4, NP, 16), jnp.float32),
    )(agg4, h4, wr, wt, b)


def _l3_call(agg4, h4, wr, wt, b, wr4, wt4):
    return pl.pallas_call(
        _l3_body,
        grid=(GRID,),
        in_specs=[_chunk_spec(), _chunk_spec(),
                  _full((64, 64)), _full((64, 64)), _full((1, 64)),
                  _full((64, 1)), _full((64, 1))],
        out_specs=[pl.BlockSpec((RB, 16), lambda i: (i, 0)),
                   pl.BlockSpec((RB, 1), lambda i: (i, 0))],
        out_shape=[jax.ShapeDtypeStruct((N, 16), jnp.float32),
                   jax.ShapeDtypeStruct((N, 1), jnp.float32)],
    )(agg4, h4, wr, wt, b, wr4, wt4)


def _fin_call(aggy2, z, b4):
    return pl.pallas_call(
        _fin_body,
        grid=(GRID,),
        in_specs=[_pair_spec(),
                  pl.BlockSpec((RB, 1), lambda i: (i, 0)),
                  _full((1, 1))],
        out_specs=pl.BlockSpec((RB, 1), lambda i: (i, 0)),
        out_shape=jax.ShapeDtypeStruct((N, 1), jnp.float32),
    )(aggy2, z, b4)


def kernel(x, edge_index, edge_weights, Wr0, br0, Wt0, Wr1, br1, Wt1,
           Wr2, br2, Wt2, Wr3, br3, Wt3, Wr4, br4, Wt4):
    src = edge_index[0]
    dst = edge_index[1]
    ew = edge_weights

    x_pad = jnp.pad(x, ((0, 0), (0, 3)))
    wr0p = jnp.pad(Wr0, ((0, 0), (0, 3))).T
    wt0p = jnp.pad(Wt0, ((0, 0), (0, 3))).T

    agg0 = _agg_split(x_pad, src, dst, ew)
    h1 = _l0_call(agg0.reshape(2, NP, 16), x_pad, wr0p, wt0p,
                  br0.reshape(1, 64))

    h = h1
    for Wr, br, Wt in ((Wr1, br1, Wt1), (Wr2, br2, Wt2)):
        agg = _agg_chunk(h.reshape(4 * NP, 16), src, dst, ew)
        h = _mid_call(agg.reshape(4, NP, 16), h, Wr.T, Wt.T,
                      br.reshape(1, 64))

    agg3 = _agg_chunk(h.reshape(4 * NP, 16), src, dst, ew)
    y_pad, z = _l3_call(agg3.reshape(4, NP, 16), h, Wr3.T, Wt3.T,
                        br3.reshape(1, 64), Wr4.T, Wt4.T)

    aggy = _agg_split(y_pad, src, dst, ew)
    out = _fin_call(aggy.reshape(2, NP, 16), z, br4.reshape(1, 1))
    return out


# speedup vs baseline: 23.5593x; 1.4772x over previous
import functools

import jax
import jax.numpy as jnp
from jax import lax
from jax.experimental import pallas as pl
from jax.experimental.pallas import tpu as pltpu
from jax.experimental.pallas import tpu_sc as plsc

N = 100000
NP = 100096
E = 3200000
NSUB = 16
B = 800
RPT = NP // NSUB
RB = 4000
GRID = N // RB

_MESH = plsc.VectorSubcoreMesh(core_axis_name="c", subcore_axis_name="s")

_SCRATCH = [
    pltpu.VMEM((B,), jnp.int32),
    pltpu.VMEM((B,), jnp.int32),
    pltpu.VMEM((B,), jnp.float32),
    pltpu.VMEM((B, 16), jnp.float32),
    pltpu.VMEM((B,), jnp.int32),
    pltpu.VMEM((B,), jnp.int32),
    pltpu.VMEM((B,), jnp.float32),
    pltpu.VMEM((B, 16), jnp.float32),
    pltpu.VMEM_SHARED((NP, 16), jnp.float32),
    pltpu.SemaphoreType.DMA,
    pltpu.SemaphoreType.DMA,
    pltpu.SemaphoreType.DMA,
    pltpu.SemaphoreType.DMA,
    pltpu.SemaphoreType.DMA,
    pltpu.SemaphoreType.DMA,
]


def _zero_fill(rows_v):
    def zi(i, c):
        rows_v[i] = jnp.zeros((16,), jnp.float32)
        return c
    lax.fori_loop(0, B, zi, 0)


def _zero_agg(rows_v, agg_sh, sid):
    for k in range(RPT // B):
        off = pl.multiple_of(sid * RPT + k * B, 8)
        pltpu.sync_copy(rows_v, agg_sh.at[pl.ds(off, B)])
    rem = RPT % B
    if rem:
        off = pl.multiple_of(sid * RPT + (RPT // B) * B, 8)
        pltpu.sync_copy(rows_v.at[pl.ds(0, rem)], agg_sh.at[pl.ds(off, rem)])


def _scale(rows_v, ew_v):
    @plsc.parallel_loop(0, B, unroll=8)
    def _srow(i):
        ewi = plsc.load_gather(
            ew_v, [jnp.broadcast_to(i, (16,)).astype(jnp.int32)])
        rows_v[i] = rows_v[i] * ewi


def _edge_pass(tbl, src_hbm, dst_hbm, ew_hbm, agg_sh, bufs, base, nblk):
    def body(g, carry):
        offs = (base + (2 * g) * B, base + (2 * g) * B + B)
        eds = []
        for (sv, dv, ev, rv, se, sg, ss), off in zip(bufs, offs):
            d_s = pltpu.async_copy(src_hbm.at[pl.ds(off, B)], sv, se)
            d_d = pltpu.async_copy(dst_hbm.at[pl.ds(off, B)], dv, se)
            d_e = pltpu.async_copy(ew_hbm.at[pl.ds(off, B)], ev, se)
            eds.append((d_s, d_d, d_e))
        gds = []
        for (sv, dv, ev, rv, se, sg, ss), (d_s, d_d, d_e) in zip(bufs, eds):
            d_s.wait()
            gds.append(pltpu.async_copy(tbl.at[sv], rv, sg))
        sds = []
        for (sv, dv, ev, rv, se, sg, ss), (d_s, d_d, d_e), gd in zip(
                bufs, eds, gds):
            d_e.wait()
            gd.wait()
            _scale(rv, ev)
            d_d.wait()
            sds.append(pltpu.async_copy(rv, agg_sh.at[dv], ss, add=True))
        for sd in sds:
            sd.wait()
        return carry
    lax.fori_loop(0, nblk // 2, body, 0)
    if nblk % 2:
        off = base + (nblk - 1) * B
        (sv, dv, ev, rv, se, sg, ss) = bufs[0]
        pltpu.sync_copy(src_hbm.at[pl.ds(off, B)], sv)
        pltpu.sync_copy(dst_hbm.at[pl.ds(off, B)], dv)
        pltpu.sync_copy(ew_hbm.at[pl.ds(off, B)], ev)
        pltpu.async_copy(tbl.at[sv], rv, sg).wait()
        _scale(rv, ev)
        pltpu.sync_copy(rv, agg_sh.at[dv], add=True)


_SC_PARAMS = pltpu.CompilerParams(needs_layout_passes=False,
                                  use_tc_tiling_on_sc=False)


@functools.partial(
    pl.kernel, mesh=_MESH,
    out_type=jax.ShapeDtypeStruct((2 * NP, 16), jnp.float32),
    scratch_types=_SCRATCH, compiler_params=_SC_PARAMS)
def _agg_split(table, src, dst, ew, out,
               sv0, dv0, ev0, rv0, sv1, dv1, ev1, rv1, agg_sh,
               se0, sg0, ss0, se1, sg1, ss1):
    cid = lax.axis_index("c")
    sid = lax.axis_index("s")
    bufs = ((sv0, dv0, ev0, rv0, se0, sg0, ss0),
            (sv1, dv1, ev1, rv1, se1, sg1, ss1))
    _zero_fill(rv0)
    _zero_agg(rv0, agg_sh, sid)
    plsc.subcore_barrier()
    wid = cid * NSUB + sid
    _edge_pass(table, src, dst, ew, agg_sh, bufs,
               base=wid * (E // 32), nblk=(E // 32) // B)
    plsc.subcore_barrier()
    soff = pl.multiple_of(sid * RPT, 8)
    doff = pl.multiple_of(cid * NP + sid * RPT, 8)
    pltpu.sync_copy(agg_sh.at[pl.ds(soff, RPT)], out.at[pl.ds(doff, RPT)])


_SCRATCH_MID = [
    pltpu.VMEM((B,), jnp.int32),
    pltpu.VMEM((B,), jnp.int32),
    pltpu.VMEM((B,), jnp.float32),
    pltpu.VMEM((B, 32), jnp.bfloat16),
    pltpu.VMEM((B,), jnp.int32),
    pltpu.VMEM((B,), jnp.int32),
    pltpu.VMEM((B,), jnp.float32),
    pltpu.VMEM((B, 32), jnp.bfloat16),
    pltpu.VMEM_SHARED((NP, 32), jnp.bfloat16),
    pltpu.SemaphoreType.DMA,
    pltpu.SemaphoreType.DMA,
    pltpu.SemaphoreType.DMA,
    pltpu.SemaphoreType.DMA,
    pltpu.SemaphoreType.DMA,
    pltpu.SemaphoreType.DMA,
]


def _zero_fill32(rows_v):
    def zi(i, c):
        rows_v[i] = jnp.zeros((32,), jnp.bfloat16)
        return c
    lax.fori_loop(0, B, zi, 0)


def _scale32(rows_v, ew_v):
    @plsc.parallel_loop(0, B, unroll=8)
    def _srow(i):
        ewi = plsc.load_gather(
            ew_v, [jnp.broadcast_to(i, (16,)).astype(jnp.int32)])
        row = rows_v[i]
        a, b2 = plsc.unpack(row, format=plsc.PackFormat.INTERLEAVED)
        rows_v[i] = plsc.pack(a * ewi, b2 * ewi,
                              format=plsc.PackFormat.INTERLEAVED)


def _edge_pass32(tbl, src_hbm, dst_hbm, ew_hbm, agg_sh, bufs, base, nblk):
    def body(g, carry):
        offs = (base + (2 * g) * B, base + (2 * g) * B + B)
        eds = []
        for (sv, dv, ev, rv, se, sg, ss), off in zip(bufs, offs):
            d_s = pltpu.async_copy(src_hbm.at[pl.ds(off, B)], sv, se)
            d_d = pltpu.async_copy(dst_hbm.at[pl.ds(off, B)], dv, se)
            d_e = pltpu.async_copy(ew_hbm.at[pl.ds(off, B)], ev, se)
            eds.append((d_s, d_d, d_e))
        gds = []
        for (sv, dv, ev, rv, se, sg, ss), (d_s, d_d, d_e) in zip(bufs, eds):
            d_s.wait()
            gds.append(pltpu.async_copy(tbl.at[sv], rv, sg))
        sds = []
        for (sv, dv, ev, rv, se, sg, ss), (d_s, d_d, d_e), gd in zip(
                bufs, eds, gds):
            d_e.wait()
            gd.wait()
            _scale32(rv, ev)
            d_d.wait()
            sds.append(pltpu.async_copy(rv, agg_sh.at[dv], ss, add=True))
        for sd in sds:
            sd.wait()
        return carry
    lax.fori_loop(0, nblk // 2, body, 0)


@functools.partial(
    pl.kernel, mesh=_MESH,
    out_type=jax.ShapeDtypeStruct((2 * NP, 32), jnp.bfloat16),
    scratch_types=_SCRATCH_MID, compiler_params=_SC_PARAMS)
def _agg_mid(table, src, dst, ew, out,
             sv0, dv0, ev0, rv0, sv1, dv1, ev1, rv1, agg_sh,
             se0, sg0, ss0, se1, sg1, ss1):
    cid = lax.axis_index("c")
    sid = lax.axis_index("s")
    bufs = ((sv0, dv0, ev0, rv0, se0, sg0, ss0),
            (sv1, dv1, ev1, rv1, se1, sg1, ss1))
    toff = pl.multiple_of(cid * NP, 8)
    tbl = table.at[pl.ds(toff, NP)]
    _zero_fill32(rv0)
    for k in range(RPT // B):
        zoff = pl.multiple_of(sid * RPT + k * B, 8)
        pltpu.sync_copy(rv0, agg_sh.at[pl.ds(zoff, B)])
    rem = RPT % B
    if rem:
        zoff = pl.multiple_of(sid * RPT + (RPT // B) * B, 8)
        pltpu.sync_copy(rv0.at[pl.ds(0, rem)], agg_sh.at[pl.ds(zoff, rem)])
    plsc.subcore_barrier()
    _edge_pass32(tbl, src, dst, ew, agg_sh, bufs,
                 base=sid * (E // 16), nblk=(E // 16) // B)
    plsc.subcore_barrier()
    soff = pl.multiple_of(sid * RPT, 8)
    doff = pl.multiple_of(cid * NP + sid * RPT, 8)
    pltpu.sync_copy(agg_sh.at[pl.ds(soff, RPT)], out.at[pl.ds(doff, RPT)])


def _l0_body(agg_r, xp_r, wr_r, wt_r, b_r, out_r):
    agg = agg_r[0] + agg_r[1]
    acc = jnp.dot(agg, wr_r[...], preferred_element_type=jnp.float32)
    acc = acc + jnp.dot(xp_r[...], wt_r[...], preferred_element_type=jnp.float32)
    acc = acc + b_r[...]
    h = jnp.maximum(acc, 0.0).astype(jnp.bfloat16)
    for c in range(2):
        out_r[c] = h[:, 32 * c:32 * (c + 1)]


def _mid_body(agg_r, h_r, wr_r, wt_r, b_r, out_r):
    a64 = jnp.concatenate([agg_r[0], agg_r[1]], axis=1).astype(jnp.float32)
    h64 = jnp.concatenate([h_r[0], h_r[1]], axis=1).astype(jnp.float32)
    acc = jnp.dot(a64, wr_r[...], preferred_element_type=jnp.float32)
    acc = acc + jnp.dot(h64, wt_r[...], preferred_element_type=jnp.float32)
    acc = acc + b_r[...]
    h = jnp.maximum(acc, 0.0).astype(jnp.bfloat16)
    for c in range(2):
        out_r[c] = h[:, 32 * c:32 * (c + 1)]


def _l3_body(agg_r, h_r, wr_r, wt_r, b_r, wr4_r, wt4_r, yp_r, z_r):
    a64 = jnp.concatenate([agg_r[0], agg_r[1]], axis=1).astype(jnp.float32)
    h64 = jnp.concatenate([h_r[0], h_r[1]], axis=1).astype(jnp.float32)
    acc = jnp.dot(a64, wr_r[...], preferred_element_type=jnp.float32)
    acc = acc + jnp.dot(h64, wt_r[...], preferred_element_type=jnp.float32)
    acc = acc + b_r[...]
    h4 = jnp.maximum(acc, 0.0)
    y = jnp.dot(h4, wr4_r[...], preferred_element_type=jnp.float32)
    z = jnp.dot(h4, wt4_r[...], preferred_element_type=jnp.float32)
    yp_r[...] = jnp.concatenate([y, jnp.zeros((RB, 15), jnp.float32)], axis=1)
    z_r[...] = z


def _fin_body(aggy_r, z_r, b_r, out_r):
    s = aggy_r[0][:, 0:1] + aggy_r[1][:, 0:1] + z_r[...] + b_r[...]
    out_r[...] = jax.nn.sigmoid(s)


def _half_spec():
    return pl.BlockSpec((2, RB, 32), lambda i: (0, i, 0))


def _pair_spec():
    return pl.BlockSpec((2, RB, 16), lambda i: (0, i, 0))


def _full(shape):
    return pl.BlockSpec(shape, lambda i: tuple(0 for _ in shape))


def _l0_call(agg2, xp, wr, wt, b):
    return pl.pallas_call(
        _l0_body,
        grid=(GRID,),
        in_specs=[_pair_spec(),
                  pl.BlockSpec((RB, 16), lambda i: (i, 0)),
                  _full((16, 64)), _full((16, 64)), _full((1, 64))],
        out_specs=_half_spec(),
        out_shape=jax.ShapeDtypeStruct((2, NP, 32), jnp.bfloat16),
    )(agg2, xp, wr, wt, b)


def _mid_call(agg4, h4, wr, wt, b):
    return pl.pallas_call(
        _mid_body,
        grid=(GRID,),
        in_specs=[_half_spec(), _half_spec(),
                  _full((64, 64)), _full((64, 64)), _full((1, 64))],
        out_specs=_half_spec(),
        out_shape=jax.ShapeDtypeStruct((2, NP, 32), jnp.bfloat16),
    )(agg4, h4, wr, wt, b)


def _l3_call(agg4, h4, wr, wt, b, wr4, wt4):
    return pl.pallas_call(
        _l3_body,
        grid=(GRID,),
        in_specs=[_half_spec(), _half_spec(),
                  _full((64, 64)), _full((64, 64)), _full((1, 64)),
                  _full((64, 1)), _full((64, 1))],
        out_specs=[pl.BlockSpec((RB, 16), lambda i: (i, 0)),
                   pl.BlockSpec((RB, 1), lambda i: (i, 0))],
        out_shape=[jax.ShapeDtypeStruct((N, 16), jnp.float32),
                   jax.ShapeDtypeStruct((N, 1), jnp.float32)],
    )(agg4, h4, wr, wt, b, wr4, wt4)


def _fin_call(aggy2, z, b4):
    return pl.pallas_call(
        _fin_body,
        grid=(GRID,),
        in_specs=[_pair_spec(),
                  pl.BlockSpec((RB, 1), lambda i: (i, 0)),
                  _full((1, 1))],
        out_specs=pl.BlockSpec((RB, 1), lambda i: (i, 0)),
        out_shape=jax.ShapeDtypeStruct((N, 1), jnp.float32),
    )(aggy2, z, b4)


def kernel(x, edge_index, edge_weights, Wr0, br0, Wt0, Wr1, br1, Wt1,
           Wr2, br2, Wt2, Wr3, br3, Wt3, Wr4, br4, Wt4):
    src = edge_index[0]
    dst = edge_index[1]
    ew = edge_weights

    x_pad = jnp.pad(x, ((0, 0), (0, 3)))
    wr0p = jnp.pad(Wr0, ((0, 0), (0, 3))).T
    wt0p = jnp.pad(Wt0, ((0, 0), (0, 3))).T

    agg0 = _agg_split(x_pad, src, dst, ew)
    h = _l0_call(agg0.reshape(2, NP, 16), x_pad, wr0p, wt0p,
                 br0.reshape(1, 64))

    for Wr, br, Wt in ((Wr1, br1, Wt1), (Wr2, br2, Wt2)):
        agg = _agg_mid(h.reshape(2 * NP, 32), src, dst, ew)
        h = _mid_call(agg.reshape(2, NP, 32), h, Wr.T, Wt.T,
                      br.reshape(1, 64))

    agg3 = _agg_mid(h.reshape(2 * NP, 32), src, dst, ew)
    y_pad, z = _l3_call(agg3.reshape(2, NP, 32), h, Wr3.T, Wt3.T,
                        br3.reshape(1, 64), Wr4.T, Wt4.T)

    aggy = _agg_split(y_pad, src, dst, ew)
    out = _fin_call(aggy.reshape(2, NP, 16), z, br4.reshape(1, 1))
    return out
